# Initial kernel scaffold; baseline (speedup 1.0000x reference)
import jax
import jax.numpy as jnp
from jax.experimental import pallas as pl


def kernel(x, Wq, bq, Wk, bk, Wv, bv, Wo, bo):
    raise NotImplementedError("write your pallas kernel here")


# speedup vs baseline: 2.2301x; 2.2301x over previous
import functools

import jax
import jax.numpy as jnp
import numpy as np
from jax import lax
from jax.experimental import pallas as pl
from jax.experimental.pallas import tpu as pltpu
from jax.experimental.pallas import tpu_sc as plsc

_B, _L, _D, _H = 2, 2048, 1024, 16
_HD = _D // _H
_BH = _B * _H
_U = max(1, _L // 5)
_UP = 416
_COMP = _L - _U
_COMPP = 4 * _UP
_CH = 104
_RT = 512
_NRT = (_B * _L) // _RT
_SCALE = 1.0 / float(np.sqrt(_HD))

_BLOCKDIAG = np.repeat(np.eye(_H, dtype=np.float32), _HD, axis=0)


def _bf16(t):
    return t.astype(jnp.bfloat16)


def _qn_body(x_ref, wq_ref, bq_ref, bd_ref, q_ref, nrm_ref):
    xb = _bf16(x_ref[...])
    q = lax.dot_general(xb, _bf16(wq_ref[...]), (((1,), (0,)), ((), ())),
                        preferred_element_type=jnp.float32) + bq_ref[...]
    q_ref[...] = q
    n2 = lax.dot_general(q * q, bd_ref[...], (((1,), (0,)), ((), ())),
                         preferred_element_type=jnp.float32,
                         precision=lax.Precision.HIGHEST)
    nrm_ref[...] = jnp.sqrt(n2)


def _qn(x2, Wq, bq):
    full = lambda i: (0, 0)
    row = lambda i: (i, 0)
    return pl.pallas_call(
        _qn_body,
        grid=(_NRT,),
        in_specs=[
            pl.BlockSpec((_RT, _D), row),
            pl.BlockSpec((_D, _D), full),
            pl.BlockSpec((1, _D), full),
            pl.BlockSpec((_D, _H), full),
        ],
        out_specs=[
            pl.BlockSpec((_RT, _D), row), pl.BlockSpec((_RT, _H), row),
        ],
        out_shape=[
            jax.ShapeDtypeStruct((_B * _L, _D), jnp.float32),
            jax.ShapeDtypeStruct((_B * _L, _H), jnp.float32),
        ],
    )(x2, Wq, bq.reshape(1, _D), jnp.asarray(_BLOCKDIAG))


def _kv_body(x_ref, wk_ref, wv_ref, bk_ref, bv_ref, k_ref, v_ref):
    xb = _bf16(x_ref[...])

    def mm(w_ref, b_ref):
        return lax.dot_general(
            xb, _bf16(w_ref[...]), (((1,), (0,)), ((), ())),
            preferred_element_type=jnp.float32) + b_ref[...]

    k_ref[...] = mm(wk_ref, bk_ref)
    v_ref[...] = mm(wv_ref, bv_ref)


def _kv(x2, Wk, Wv, bk, bv):
    full = lambda i: (0, 0)
    row = lambda i: (i, 0)
    return pl.pallas_call(
        _kv_body,
        grid=(_NRT,),
        in_specs=[
            pl.BlockSpec((_RT, _D), row),
            pl.BlockSpec((_D, _D), full), pl.BlockSpec((_D, _D), full),
            pl.BlockSpec((1, _D), full), pl.BlockSpec((1, _D), full),
        ],
        out_specs=[
            pl.BlockSpec((_RT, _D), row), pl.BlockSpec((_RT, _D), row),
        ],
        out_shape=[
            jax.ShapeDtypeStruct((_B * _L, _D), jnp.float32),
            jax.ShapeDtypeStruct((_B * _L, _D), jnp.float32),
        ],
    )(x2, Wk, Wv, bk.reshape(1, _D), bv.reshape(1, _D))


def _select_gather(norms, qflat):
    mesh = plsc.VectorSubcoreMesh(core_axis_name="c", subcore_axis_name="s")

    @functools.partial(
        pl.kernel,
        mesh=mesh,
        out_type=[
            jax.ShapeDtypeStruct((_BH, _UP, _HD), jnp.float32),
            jax.ShapeDtypeStruct((_BH, _UP), jnp.int32),
            jax.ShapeDtypeStruct((_BH, _COMPP), jnp.int32),
        ],
        scratch_types=[
            pltpu.VMEM((_L, _H), jnp.float32),
            pltpu.VMEM((_L,), jnp.float32),
            pltpu.VMEM((_L + 16,), jnp.int32),
            pltpu.VMEM((_UP + 48,), jnp.int32),
            pltpu.VMEM((_COMPP + 32,), jnp.int32),
            pltpu.VMEM((_UP, _HD), jnp.float32),
            pltpu.SemaphoreType.DMA,
        ],
        compiler_params=pltpu.CompilerParams(use_tc_tiling_on_sc=False,
                                             needs_layout_passes=False),
    )
    def sel_kernel(norms_hbm, qflat_hbm, qsel_hbm, dsel_hbm, dcomp_hbm,
                   nv, ncol, candv, dselv, dcompv, qselv, sem):
        cid = lax.axis_index("c")
        sid = lax.axis_index("s")
        w = cid * 16 + sid
        b = w // _H
        h = w % _H

        pltpu.sync_copy(norms_hbm.at[pl.ds(b * _L, _L)], nv)

        iota16 = lax.iota(jnp.int32, 16)
        hv = jnp.broadcast_to(h, (16,))

        def _ext(j, c):
            rows = j * 16 + iota16
            ncol[pl.ds(j * 16, 16)] = plsc.load_gather(nv, [rows, hv])
            return c
        lax.fori_loop(0, _L // 16, _ext, jnp.int32(0))

        nvecs = _L // 16

        def _bits(j):
            return plsc.bitcast(ncol[pl.ds(j * 16, 16)], jnp.int32)

        def bit_step_a(i, pn):
            prefix, need = pn
            k = 31 - i
            m2 = lax.shift_left(jnp.int32(-1), k)
            p2 = prefix | lax.shift_left(jnp.int32(1), k)

            def cstep(j, cv):
                return cv + ((_bits(j) & m2) == p2).astype(jnp.int32)
            cv = lax.fori_loop(0, nvecs, cstep, jnp.zeros((16,), jnp.int32))
            cnt1 = jnp.sum(cv)
            take = need <= cnt1
            return (jnp.where(take, p2, prefix),
                    jnp.where(take, need, need - cnt1))

        prefix, need = lax.fori_loop(
            0, 14, bit_step_a, (jnp.int32(0), jnp.int32(_U)))

        mhi = lax.shift_left(jnp.int32(-1), jnp.int32(18))

        def comp_step(j, cnt):
            v = _bits(j)
            m = (v & mhi) == prefix
            plsc.store_compressed(candv.at[pl.ds(cnt, 16)], v, mask=m)
            return cnt + jnp.sum(m.astype(jnp.int32))
        n1 = lax.fori_loop(0, nvecs, comp_step, jnp.int32(0))
        nv1 = (n1 + 15) // 16

        def bit_step_b(i, pn):
            prefix, need = pn
            k = 17 - i
            m2 = lax.shift_left(jnp.int32(-1), k)
            p2 = prefix | lax.shift_left(jnp.int32(1), k)

            def cstep(j, cv):
                v = candv[pl.ds(j * 16, 16)]
                inb = (j * 16 + iota16) < n1
                return cv + (((v & m2) == p2) & inb).astype(jnp.int32)
            cv = lax.fori_loop(0, nv1, cstep, jnp.zeros((16,), jnp.int32))
            cnt1 = jnp.sum(cv)
            take = need <= cnt1
            return (jnp.where(take, p2, prefix),
                    jnp.where(take, need, need - cnt1))

        thresh, needf = lax.fori_loop(0, 18, bit_step_b, (prefix, need))

        base = b * (_L * _H) + h

        def fin_step(j, carry):
            off, coff, eqc = carry
            v = _bits(j)
            gt = v > thresh
            eq = v == thresh
            eqi = eq.astype(jnp.int32)
            rank = eqc + jnp.cumsum(eqi) - eqi
            take = gt | (eq & (rank < needf))
            dest = base + (j * 16 + iota16) * _H
            plsc.store_compressed(dselv.at[pl.ds(off, 16)], dest, mask=take)
            ntake = jnp.logical_not(take)
            plsc.store_compressed(dcompv.at[pl.ds(coff, 16)], dest, mask=ntake)
            return (off + jnp.sum(take.astype(jnp.int32)),
                    coff + jnp.sum(ntake.astype(jnp.int32)),
                    eqc + jnp.sum(eqi))
        lax.fori_loop(0, nvecs, fin_step,
                      (jnp.int32(0), jnp.int32(0), jnp.int32(0)))

        d0 = jnp.sum(jnp.where(iota16 == 0, dselv[pl.ds(0, 16)], 0))
        wv = dselv[pl.ds(400, 16)]
        dselv[pl.ds(400, 16)] = jnp.where(400 + iota16 < _U, wv, d0)
        c0 = jnp.sum(jnp.where(iota16 == 0, dcompv[pl.ds(0, 16)], 0))
        for cbase in (1632, 1648):
            cv = dcompv[pl.ds(cbase, 16)]
            dcompv[pl.ds(cbase, 16)] = jnp.where(cbase + iota16 < _COMP, cv, c0)

        for j in range(4):
            pltpu.async_copy(qflat_hbm.at[dselv.at[pl.ds(j * _CH, _CH)]],
                             qselv.at[pl.ds(j * _CH, _CH)], sem).wait()
        pltpu.sync_copy(qselv, qsel_hbm.at[w])
        pltpu.sync_copy(dselv.at[pl.ds(0, _UP)], dsel_hbm.at[w])
        pltpu.sync_copy(dcompv.at[pl.ds(0, _COMPP)], dcomp_hbm.at[w])

    return sel_kernel(norms, qflat)


def _attn_body(qsel_ref, k_ref, v_ref, o_ref):
    for h in range(_H):
        q = _bf16(qsel_ref[h])
        kh = _bf16(k_ref[:, h * _HD:(h + 1) * _HD])
        s = lax.dot_general(q, kh, (((1,), (1,)), ((), ())),
                            preferred_element_type=jnp.float32) * _SCALE
        m = jnp.max(s, axis=1, keepdims=True)
        e = jnp.exp(s - m)
        p = e / jnp.sum(e, axis=1, keepdims=True)
        vh = _bf16(v_ref[:, h * _HD:(h + 1) * _HD])
        o_ref[h] = lax.dot_general(_bf16(p), vh, (((1,), (0,)), ((), ())),
                                   preferred_element_type=jnp.float32)


def _attention(qsel, k2, v2):
    return pl.pallas_call(
        _attn_body,
        grid=(_B,),
        in_specs=[
            pl.BlockSpec((_H, _UP, _HD), lambda i: (i, 0, 0)),
            pl.BlockSpec((_L, _D), lambda i: (i, 0)),
            pl.BlockSpec((_L, _D), lambda i: (i, 0)),
        ],
        out_specs=pl.BlockSpec((_H, _UP, _HD), lambda i: (i, 0, 0)),
        out_shape=jax.ShapeDtypeStruct((_BH, _UP, _HD), jnp.float32),
    )(qsel, k2, v2)


def _scatter(osel, dsel, dcomp4, zrows):
    mesh = plsc.VectorSubcoreMesh(core_axis_name="c", subcore_axis_name="s")

    @functools.partial(
        pl.kernel,
        mesh=mesh,
        out_type=jax.ShapeDtypeStruct((_B * _L * _H, _HD), jnp.float32),
        scratch_types=[
            pltpu.VMEM((_UP, _HD), jnp.float32),
            pltpu.VMEM((_CH, _HD), jnp.float32),
            pltpu.VMEM((4, _CH), jnp.int32),
            pltpu.VMEM((16, _CH), jnp.int32),
            pltpu.SemaphoreType.DMA,
        ],
        compiler_params=pltpu.CompilerParams(use_tc_tiling_on_sc=False,
                                             needs_layout_passes=False),
    )
    def scat_kernel(osel_hbm, dsel_hbm, dcomp_hbm, z_hbm, out_hbm,
                    ov, zv, dv, cv, sem):
        cid = lax.axis_index("c")
        sid = lax.axis_index("s")
        w = cid * 16 + sid
        pltpu.sync_copy(osel_hbm.at[w], ov)
        pltpu.sync_copy(z_hbm, zv)
        pltpu.sync_copy(dsel_hbm.at[pl.ds(w * 4, 4)], dv)
        pltpu.sync_copy(dcomp_hbm.at[pl.ds(w * 16, 16)], cv)
        for j in range(4):
            pltpu.async_copy(ov.at[pl.ds(j * _CH, _CH)],
                             out_hbm.at[dv.at[j]], sem).wait()
        for j in range(16):
            pltpu.async_copy(zv, out_hbm.at[cv.at[j]], sem).wait()

    return scat_kernel(osel, dsel, dcomp4, zrows)


def _proj_body(y_ref, wo_ref, bo_ref, o_ref):
    o_ref[...] = lax.dot_general(
        _bf16(y_ref[...]), _bf16(wo_ref[...]), (((1,), (0,)), ((), ())),
        preferred_element_type=jnp.float32) + bo_ref[...]


def _proj(y2, Wo, bo):
    return pl.pallas_call(
        _proj_body,
        grid=(_NRT,),
        in_specs=[
            pl.BlockSpec((_RT, _D), lambda i: (i, 0)),
            pl.BlockSpec((_D, _D), lambda i: (0, 0)),
            pl.BlockSpec((1, _D), lambda i: (0, 0)),
        ],
        out_specs=pl.BlockSpec((_RT, _D), lambda i: (i, 0)),
        out_shape=jax.ShapeDtypeStruct((_B * _L, _D), jnp.float32),
    )(y2, Wo, bo.reshape(1, _D))


def kernel(x, Wq, bq, Wk, bk, Wv, bv, Wo, bo):
    x2 = x.reshape(_B * _L, _D)
    q2, norms = _qn(x2, Wq, bq)
    qflat = q2.reshape(_B * _L * _H, _HD)
    qsel, dsel, dcomp = _select_gather(norms, qflat)
    k2, v2 = _kv(x2, Wk, Wv, bk, bv)
    osel = _attention(qsel, k2, v2)
    zrows = jnp.zeros((_CH, _HD), jnp.float32)
    outflat = _scatter(osel, dsel.reshape(_BH * 4, _CH),
                       dcomp.reshape(_BH * 16, _CH), zrows)
    out2 = _proj(outflat.reshape(_B * _L, _D), Wo, bo)
    return out2.reshape(_B, _L, _D)


# speedup vs baseline: 2.2432x; 1.0059x over previous
import functools

import jax
import jax.numpy as jnp
import numpy as np
from jax import lax
from jax.experimental import pallas as pl
from jax.experimental.pallas import tpu as pltpu
from jax.experimental.pallas import tpu_sc as plsc

_B, _L, _D, _H = 2, 2048, 1024, 16
_HD = _D // _H
_BH = _B * _H
_U = max(1, _L // 5)
_UP = 416
_COMP = _L - _U
_COMPP = 4 * _UP
_CH = 104
_RT = 512
_NRT = (_B * _L) // _RT
_SCALE = 1.0 / float(np.sqrt(_HD))

_BLOCKDIAG = np.repeat(np.eye(_H, dtype=np.float32), _HD, axis=0)


def _bf16(t):
    return t.astype(jnp.bfloat16)


def _qn_body(x_ref, wq_ref, bq_ref, bd_ref, q_ref, nrm_ref):
    xb = _bf16(x_ref[...])
    q = lax.dot_general(xb, _bf16(wq_ref[...]), (((1,), (0,)), ((), ())),
                        preferred_element_type=jnp.float32) + bq_ref[...]
    q_ref[...] = q
    n2 = lax.dot_general(q * q, bd_ref[...], (((1,), (0,)), ((), ())),
                         preferred_element_type=jnp.float32,
                         precision=lax.Precision.HIGHEST)
    nrm_ref[...] = jnp.sqrt(n2)


def _qn(x2, Wq, bq):
    full = lambda i: (0, 0)
    row = lambda i: (i, 0)
    return pl.pallas_call(
        _qn_body,
        grid=(_NRT,),
        in_specs=[
            pl.BlockSpec((_RT, _D), row),
            pl.BlockSpec((_D, _D), full),
            pl.BlockSpec((1, _D), full),
            pl.BlockSpec((_D, _H), full),
        ],
        out_specs=[
            pl.BlockSpec((_RT, _D), row), pl.BlockSpec((_RT, _H), row),
        ],
        out_shape=[
            jax.ShapeDtypeStruct((_B * _L, _D), jnp.float32),
            jax.ShapeDtypeStruct((_B * _L, _H), jnp.float32),
        ],
    )(x2, Wq, bq.reshape(1, _D), jnp.asarray(_BLOCKDIAG))


def _kv_body(x_ref, wk_ref, wv_ref, bk_ref, bv_ref, k_ref, v_ref):
    xb = _bf16(x_ref[...])

    def mm(w_ref, b_ref):
        return lax.dot_general(
            xb, _bf16(w_ref[...]), (((1,), (0,)), ((), ())),
            preferred_element_type=jnp.float32) + b_ref[...]

    k_ref[...] = mm(wk_ref, bk_ref)
    v_ref[...] = mm(wv_ref, bv_ref)


def _kv(x2, Wk, Wv, bk, bv):
    full = lambda i: (0, 0)
    row = lambda i: (i, 0)
    return pl.pallas_call(
        _kv_body,
        grid=(_NRT,),
        in_specs=[
            pl.BlockSpec((_RT, _D), row),
            pl.BlockSpec((_D, _D), full), pl.BlockSpec((_D, _D), full),
            pl.BlockSpec((1, _D), full), pl.BlockSpec((1, _D), full),
        ],
        out_specs=[
            pl.BlockSpec((_RT, _D), row), pl.BlockSpec((_RT, _D), row),
        ],
        out_shape=[
            jax.ShapeDtypeStruct((_B * _L, _D), jnp.float32),
            jax.ShapeDtypeStruct((_B * _L, _D), jnp.float32),
        ],
    )(x2, Wk, Wv, bk.reshape(1, _D), bv.reshape(1, _D))


def _select_gather(norms, qflat):
    mesh = plsc.VectorSubcoreMesh(core_axis_name="c", subcore_axis_name="s")

    @functools.partial(
        pl.kernel,
        mesh=mesh,
        out_type=[
            jax.ShapeDtypeStruct((_BH, _UP, _HD), jnp.float32),
            jax.ShapeDtypeStruct((_BH, _UP), jnp.int32),
            jax.ShapeDtypeStruct((_BH, _COMPP), jnp.int32),
        ],
        scratch_types=[
            pltpu.VMEM((_L, _H), jnp.float32),
            pltpu.VMEM((_L,), jnp.float32),
            pltpu.VMEM((_L + 16,), jnp.int32),
            pltpu.VMEM((_UP + 48,), jnp.int32),
            pltpu.VMEM((_COMPP + 32,), jnp.int32),
            pltpu.VMEM((_UP, _HD), jnp.float32),
            pltpu.SemaphoreType.DMA,
        ],
        compiler_params=pltpu.CompilerParams(use_tc_tiling_on_sc=False,
                                             needs_layout_passes=False),
    )
    def sel_kernel(norms_hbm, qflat_hbm, qsel_hbm, dsel_hbm, dcomp_hbm,
                   nv, ncol, candv, dselv, dcompv, qselv, sem):
        cid = lax.axis_index("c")
        sid = lax.axis_index("s")
        w = cid * 16 + sid
        b = w // _H
        h = w % _H

        pltpu.sync_copy(norms_hbm.at[pl.ds(b * _L, _L)], nv)

        iota16 = lax.iota(jnp.int32, 16)
        hv = jnp.broadcast_to(h, (16,))

        def _ext(j, mm_):
            mnv, mxv = mm_
            rows = j * 16 + iota16
            vals = plsc.load_gather(nv, [rows, hv])
            ncol[pl.ds(j * 16, 16)] = vals
            bits = plsc.bitcast(vals, jnp.int32)
            return (jnp.minimum(mnv, bits), jnp.maximum(mxv, bits))
        mnv, mxv = lax.fori_loop(
            0, _L // 16, _ext,
            (jnp.full((16,), jnp.int32(2147483647)),
             jnp.zeros((16,), jnp.int32)))
        mn_s = jnp.min(mnv)
        diff = mn_s ^ jnp.max(mxv)

        nvecs = _L // 16

        def _bits(j):
            return plsc.bitcast(ncol[pl.ds(j * 16, 16)], jnp.int32)

        def bit_step_a(i, pn):
            prefix, need = pn
            k = 31 - i
            bitk = lax.shift_left(jnp.int32(1), k)
            m2 = lax.shift_left(jnp.int32(-1), k)
            p2 = prefix | bitk

            def skip_common(_):
                return (prefix | (mn_s & bitk), need)

            def do_count(_):
                def cstep(j, cv):
                    return cv + ((_bits(j) & m2) == p2).astype(jnp.int32)
                cv = lax.fori_loop(0, nvecs, cstep,
                                   jnp.zeros((16,), jnp.int32))
                cnt1 = jnp.sum(cv)
                take = need <= cnt1
                return (jnp.where(take, p2, prefix),
                        jnp.where(take, need, need - cnt1))

            return lax.cond(lax.shift_right_logical(diff, k) == 0,
                            skip_common, do_count, 0)

        prefix, need = lax.fori_loop(
            0, 14, bit_step_a, (jnp.int32(0), jnp.int32(_U)))

        mhi = lax.shift_left(jnp.int32(-1), jnp.int32(18))

        def comp_step(j, cnt):
            v = _bits(j)
            m = (v & mhi) == prefix
            plsc.store_compressed(candv.at[pl.ds(cnt, 16)], v, mask=m)
            return cnt + jnp.sum(m.astype(jnp.int32))
        n1 = lax.fori_loop(0, nvecs, comp_step, jnp.int32(0))
        nv1 = (n1 + 15) // 16

        def bit_step_b(i, pn):
            prefix, need = pn
            k = 17 - i
            m2 = lax.shift_left(jnp.int32(-1), k)
            p2 = prefix | lax.shift_left(jnp.int32(1), k)

            def cstep(j, cv):
                v = candv[pl.ds(j * 16, 16)]
                inb = (j * 16 + iota16) < n1
                return cv + (((v & m2) == p2) & inb).astype(jnp.int32)
            cv = lax.fori_loop(0, nv1, cstep, jnp.zeros((16,), jnp.int32))
            cnt1 = jnp.sum(cv)
            take = need <= cnt1
            return (jnp.where(take, p2, prefix),
                    jnp.where(take, need, need - cnt1))

        thresh, needf = lax.fori_loop(0, 18, bit_step_b, (prefix, need))

        base = b * (_L * _H) + h

        def fin_step(j, carry):
            off, coff, eqc = carry
            v = _bits(j)
            gt = v > thresh
            eq = v == thresh
            eqi = eq.astype(jnp.int32)
            rank = eqc + jnp.cumsum(eqi) - eqi
            take = gt | (eq & (rank < needf))
            dest = base + (j * 16 + iota16) * _H
            plsc.store_compressed(dselv.at[pl.ds(off, 16)], dest, mask=take)
            ntake = jnp.logical_not(take)
            plsc.store_compressed(dcompv.at[pl.ds(coff, 16)], dest, mask=ntake)
            return (off + jnp.sum(take.astype(jnp.int32)),
                    coff + jnp.sum(ntake.astype(jnp.int32)),
                    eqc + jnp.sum(eqi))
        lax.fori_loop(0, nvecs, fin_step,
                      (jnp.int32(0), jnp.int32(0), jnp.int32(0)))

        d0 = jnp.sum(jnp.where(iota16 == 0, dselv[pl.ds(0, 16)], 0))
        wv = dselv[pl.ds(400, 16)]
        dselv[pl.ds(400, 16)] = jnp.where(400 + iota16 < _U, wv, d0)
        c0 = jnp.sum(jnp.where(iota16 == 0, dcompv[pl.ds(0, 16)], 0))
        for cbase in (1632, 1648):
            cv = dcompv[pl.ds(cbase, 16)]
            dcompv[pl.ds(cbase, 16)] = jnp.where(cbase + iota16 < _COMP, cv, c0)

        for j in range(4):
            pltpu.async_copy(qflat_hbm.at[dselv.at[pl.ds(j * _CH, _CH)]],
                             qselv.at[pl.ds(j * _CH, _CH)], sem).wait()
        pltpu.sync_copy(qselv, qsel_hbm.at[w])
        pltpu.sync_copy(dselv.at[pl.ds(0, _UP)], dsel_hbm.at[w])
        pltpu.sync_copy(dcompv.at[pl.ds(0, _COMPP)], dcomp_hbm.at[w])

    return sel_kernel(norms, qflat)


def _attn_body(qsel_ref, k_ref, v_ref, o_ref):
    for h in range(_H):
        q = _bf16(qsel_ref[h])
        kh = _bf16(k_ref[:, h * _HD:(h + 1) * _HD])
        s = lax.dot_general(q, kh, (((1,), (1,)), ((), ())),
                            preferred_element_type=jnp.float32) * _SCALE
        m = jnp.max(s, axis=1, keepdims=True)
        e = jnp.exp(_bf16(s - m))
        denom = jnp.sum(e.astype(jnp.float32), axis=1, keepdims=True)
        p = e * _bf16(1.0 / denom)
        vh = _bf16(v_ref[:, h * _HD:(h + 1) * _HD])
        o_ref[h] = lax.dot_general(p, vh, (((1,), (0,)), ((), ())),
                                   preferred_element_type=jnp.float32)


def _attention(qsel, k2, v2):
    return pl.pallas_call(
        _attn_body,
        grid=(_B,),
        in_specs=[
            pl.BlockSpec((_H, _UP, _HD), lambda i: (i, 0, 0)),
            pl.BlockSpec((_L, _D), lambda i: (i, 0)),
            pl.BlockSpec((_L, _D), lambda i: (i, 0)),
        ],
        out_specs=pl.BlockSpec((_H, _UP, _HD), lambda i: (i, 0, 0)),
        out_shape=jax.ShapeDtypeStruct((_BH, _UP, _HD), jnp.float32),
    )(qsel, k2, v2)


def _scatter(osel, dsel, dcomp4, zrows):
    mesh = plsc.VectorSubcoreMesh(core_axis_name="c", subcore_axis_name="s")

    @functools.partial(
        pl.kernel,
        mesh=mesh,
        out_type=jax.ShapeDtypeStruct((_B * _L * _H, _HD), jnp.float32),
        scratch_types=[
            pltpu.VMEM((_UP, _HD), jnp.float32),
            pltpu.VMEM((_CH, _HD), jnp.float32),
            pltpu.VMEM((4, _CH), jnp.int32),
            pltpu.VMEM((16, _CH), jnp.int32),
            pltpu.SemaphoreType.DMA,
        ],
        compiler_params=pltpu.CompilerParams(use_tc_tiling_on_sc=False,
                                             needs_layout_passes=False),
    )
    def scat_kernel(osel_hbm, dsel_hbm, dcomp_hbm, z_hbm, out_hbm,
                    ov, zv, dv, cv, sem):
        cid = lax.axis_index("c")
        sid = lax.axis_index("s")
        w = cid * 16 + sid
        pltpu.sync_copy(osel_hbm.at[w], ov)
        pltpu.sync_copy(z_hbm, zv)
        pltpu.sync_copy(dsel_hbm.at[pl.ds(w * 4, 4)], dv)
        pltpu.sync_copy(dcomp_hbm.at[pl.ds(w * 16, 16)], cv)
        for j in range(4):
            pltpu.async_copy(ov.at[pl.ds(j * _CH, _CH)],
                             out_hbm.at[dv.at[j]], sem).wait()
        for j in range(16):
            pltpu.async_copy(zv, out_hbm.at[cv.at[j]], sem).wait()

    return scat_kernel(osel, dsel, dcomp4, zrows)


def _proj_body(y_ref, wo_ref, bo_ref, o_ref):
    o_ref[...] = lax.dot_general(
        _bf16(y_ref[...]), _bf16(wo_ref[...]), (((1,), (0,)), ((), ())),
        preferred_element_type=jnp.float32) + bo_ref[...]


def _proj(y2, Wo, bo):
    return pl.pallas_call(
        _proj_body,
        grid=(_NRT,),
        in_specs=[
            pl.BlockSpec((_RT, _D), lambda i: (i, 0)),
            pl.BlockSpec((_D, _D), lambda i: (0, 0)),
            pl.BlockSpec((1, _D), lambda i: (0, 0)),
        ],
        out_specs=pl.BlockSpec((_RT, _D), lambda i: (i, 0)),
        out_shape=jax.ShapeDtypeStruct((_B * _L, _D), jnp.float32),
    )(y2, Wo, bo.reshape(1, _D))


def kernel(x, Wq, bq, Wk, bk, Wv, bv, Wo, bo):
    x2 = x.reshape(_B * _L, _D)
    q2, norms = _qn(x2, Wq, bq)
    qflat = q2.reshape(_B * _L * _H, _HD)
    qsel, dsel, dcomp = _select_gather(norms, qflat)
    k2, v2 = _kv(x2, Wk, Wv, bk, bv)
    osel = _attention(qsel, k2, v2)
    zrows = jnp.zeros((_CH, _HD), jnp.float32)
    outflat = _scatter(osel, dsel.reshape(_BH * 4, _CH),
                       dcomp.reshape(_BH * 16, _CH), zrows)
    out2 = _proj(outflat.reshape(_B * _L, _D), Wo, bo)
    return out2.reshape(_B, _L, _D)


# speedup vs baseline: 2.2623x; 1.0085x over previous
import functools

import jax
import jax.numpy as jnp
import numpy as np
from jax import lax
from jax.experimental import pallas as pl
from jax.experimental.pallas import tpu as pltpu
from jax.experimental.pallas import tpu_sc as plsc

_B, _L, _D, _H = 2, 2048, 1024, 16
_HD = _D // _H
_BH = _B * _H
_U = max(1, _L // 5)
_UP = 416
_COMP = _L - _U
_COMPP = 4 * _UP
_CH = 104
_RT = 512
_NRT = (_B * _L) // _RT
_SCALE = 1.0 / float(np.sqrt(_HD))

_BLOCKDIAG = np.repeat(np.eye(_H, dtype=np.float32), _HD, axis=0)


def _bf16(t):
    return t.astype(jnp.bfloat16)


def _qn_body(x_ref, wq_ref, bq_ref, bd_ref, q_ref, nrm_ref):
    xb = _bf16(x_ref[...])
    q = lax.dot_general(xb, _bf16(wq_ref[...]), (((1,), (0,)), ((), ())),
                        preferred_element_type=jnp.float32) + bq_ref[...]
    q_ref[...] = q
    n2 = lax.dot_general(q * q, bd_ref[...], (((1,), (0,)), ((), ())),
                         preferred_element_type=jnp.float32,
                         precision=lax.Precision.HIGHEST)
    nrm_ref[...] = jnp.sqrt(n2)


def _qn(x2, Wq, bq):
    full = lambda i: (0, 0)
    row = lambda i: (i, 0)
    return pl.pallas_call(
        _qn_body,
        grid=(_NRT,),
        in_specs=[
            pl.BlockSpec((_RT, _D), row),
            pl.BlockSpec((_D, _D), full),
            pl.BlockSpec((1, _D), full),
            pl.BlockSpec((_D, _H), full),
        ],
        out_specs=[
            pl.BlockSpec((_RT, _D), row), pl.BlockSpec((_RT, _H), row),
        ],
        out_shape=[
            jax.ShapeDtypeStruct((_B * _L, _D), jnp.float32),
            jax.ShapeDtypeStruct((_B * _L, _H), jnp.float32),
        ],
    )(x2, Wq, bq.reshape(1, _D), jnp.asarray(_BLOCKDIAG))


def _kv_body(x_ref, wk_ref, wv_ref, bk_ref, bv_ref, k_ref, v_ref):
    xb = _bf16(x_ref[...])

    def mm(w_ref, b_ref):
        return lax.dot_general(
            xb, _bf16(w_ref[...]), (((1,), (0,)), ((), ())),
            preferred_element_type=jnp.float32) + b_ref[...]

    k_ref[...] = mm(wk_ref, bk_ref)
    v_ref[...] = mm(wv_ref, bv_ref)


def _kv(x2, Wk, Wv, bk, bv):
    full = lambda i: (0, 0)
    row = lambda i: (i, 0)
    return pl.pallas_call(
        _kv_body,
        grid=(_NRT,),
        in_specs=[
            pl.BlockSpec((_RT, _D), row),
            pl.BlockSpec((_D, _D), full), pl.BlockSpec((_D, _D), full),
            pl.BlockSpec((1, _D), full), pl.BlockSpec((1, _D), full),
        ],
        out_specs=[
            pl.BlockSpec((_RT, _D), row), pl.BlockSpec((_RT, _D), row),
        ],
        out_shape=[
            jax.ShapeDtypeStruct((_B * _L, _D), jnp.float32),
            jax.ShapeDtypeStruct((_B * _L, _D), jnp.float32),
        ],
    )(x2, Wk, Wv, bk.reshape(1, _D), bv.reshape(1, _D))


def _select_gather(norms, qflat):
    mesh = plsc.VectorSubcoreMesh(core_axis_name="c", subcore_axis_name="s")

    @functools.partial(
        pl.kernel,
        mesh=mesh,
        out_type=[
            jax.ShapeDtypeStruct((_BH, _UP, _HD), jnp.float32),
            jax.ShapeDtypeStruct((_BH, _UP), jnp.int32),
            jax.ShapeDtypeStruct((_BH, _COMPP), jnp.int32),
        ],
        scratch_types=[
            pltpu.VMEM((_L, _H), jnp.float32),
            pltpu.VMEM((_L,), jnp.float32),
            pltpu.VMEM((_L + 16,), jnp.int32),
            pltpu.VMEM((_UP + 48,), jnp.int32),
            pltpu.VMEM((_COMPP + 32,), jnp.int32),
            pltpu.VMEM((_UP, _HD), jnp.float32),
            pltpu.SemaphoreType.DMA,
        ],
        compiler_params=pltpu.CompilerParams(use_tc_tiling_on_sc=False,
                                             needs_layout_passes=False),
    )
    def sel_kernel(norms_hbm, qflat_hbm, qsel_hbm, dsel_hbm, dcomp_hbm,
                   nv, ncol, candv, dselv, dcompv, qselv, sem):
        cid = lax.axis_index("c")
        sid = lax.axis_index("s")
        w = cid * 16 + sid
        b = w // _H
        h = w % _H

        pltpu.sync_copy(norms_hbm.at[pl.ds(b * _L, _L)], nv)

        iota16 = lax.iota(jnp.int32, 16)
        hv = jnp.broadcast_to(h, (16,))

        def _ext(j, mm_):
            mnv, mxv = mm_
            rows = j * 16 + iota16
            vals = plsc.load_gather(nv, [rows, hv])
            ncol[pl.ds(j * 16, 16)] = vals
            bits = plsc.bitcast(vals, jnp.int32)
            return (jnp.minimum(mnv, bits), jnp.maximum(mxv, bits))
        mnv, mxv = lax.fori_loop(
            0, _L // 16, _ext,
            (jnp.full((16,), jnp.int32(2147483647)),
             jnp.zeros((16,), jnp.int32)))
        mn_s = jnp.min(mnv)
        diff = mn_s ^ jnp.max(mxv)

        nvecs = _L // 16

        def _bits(j):
            return plsc.bitcast(ncol[pl.ds(j * 16, 16)], jnp.int32)

        def bit_step_a(i, pn):
            prefix, need = pn
            k = 31 - i
            bitk = lax.shift_left(jnp.int32(1), k)
            m2 = lax.shift_left(jnp.int32(-1), k)
            p2 = prefix | bitk

            def skip_common(_):
                return (prefix | (mn_s & bitk), need)

            def do_count(_):
                def cstep(j, cv):
                    return cv + ((_bits(j) & m2) == p2).astype(jnp.int32)
                cv = lax.fori_loop(0, nvecs, cstep,
                                   jnp.zeros((16,), jnp.int32))
                cnt1 = jnp.sum(cv)
                take = need <= cnt1
                return (jnp.where(take, p2, prefix),
                        jnp.where(take, need, need - cnt1))

            return lax.cond(lax.shift_right_logical(diff, k) == 0,
                            skip_common, do_count, 0)

        prefix, need = lax.fori_loop(
            0, 14, bit_step_a, (jnp.int32(0), jnp.int32(_U)))

        mhi = lax.shift_left(jnp.int32(-1), jnp.int32(18))

        def comp_step(j, cnt):
            v = _bits(j)
            m = (v & mhi) == prefix
            plsc.store_compressed(candv.at[pl.ds(cnt, 16)], v, mask=m)
            return cnt + jnp.sum(m.astype(jnp.int32))
        n1 = lax.fori_loop(0, nvecs, comp_step, jnp.int32(0))
        nv1 = (n1 + 15) // 16

        def bit_step_b(i, pn):
            prefix, need = pn
            k = 17 - i
            m2 = lax.shift_left(jnp.int32(-1), k)
            p2 = prefix | lax.shift_left(jnp.int32(1), k)

            def cstep(j, cv):
                v = candv[pl.ds(j * 16, 16)]
                inb = (j * 16 + iota16) < n1
                return cv + (((v & m2) == p2) & inb).astype(jnp.int32)
            cv = lax.fori_loop(0, nv1, cstep, jnp.zeros((16,), jnp.int32))
            cnt1 = jnp.sum(cv)
            take = need <= cnt1
            return (jnp.where(take, p2, prefix),
                    jnp.where(take, need, need - cnt1))

        thresh, needf = lax.fori_loop(0, 18, bit_step_b, (prefix, need))

        base = b * (_L * _H) + h

        def fin_step(j, carry):
            off, coff, eqc = carry
            v = _bits(j)
            gt = v > thresh
            eq = v == thresh
            eqi = eq.astype(jnp.int32)
            rank = eqc + jnp.cumsum(eqi) - eqi
            take = gt | (eq & (rank < needf))
            dest = base + (j * 16 + iota16) * _H
            plsc.store_compressed(dselv.at[pl.ds(off, 16)], dest, mask=take)
            ntake = jnp.logical_not(take)
            plsc.store_compressed(dcompv.at[pl.ds(coff, 16)], dest, mask=ntake)
            return (off + jnp.sum(take.astype(jnp.int32)),
                    coff + jnp.sum(ntake.astype(jnp.int32)),
                    eqc + jnp.sum(eqi))
        lax.fori_loop(0, nvecs, fin_step,
                      (jnp.int32(0), jnp.int32(0), jnp.int32(0)))

        d0 = jnp.sum(jnp.where(iota16 == 0, dselv[pl.ds(0, 16)], 0))
        wv = dselv[pl.ds(400, 16)]
        dselv[pl.ds(400, 16)] = jnp.where(400 + iota16 < _U, wv, d0)
        c0 = jnp.sum(jnp.where(iota16 == 0, dcompv[pl.ds(0, 16)], 0))
        for cbase in (1632, 1648):
            cv = dcompv[pl.ds(cbase, 16)]
            dcompv[pl.ds(cbase, 16)] = jnp.where(cbase + iota16 < _COMP, cv, c0)

        for j in range(4):
            pltpu.async_copy(qflat_hbm.at[dselv.at[pl.ds(j * _CH, _CH)]],
                             qselv.at[pl.ds(j * _CH, _CH)], sem).wait()
        pltpu.sync_copy(qselv, qsel_hbm.at[w])
        pltpu.sync_copy(dselv.at[pl.ds(0, _UP)], dsel_hbm.at[w])
        pltpu.sync_copy(dcompv.at[pl.ds(0, _COMPP)], dcomp_hbm.at[w])

    return sel_kernel(norms, qflat)


def _attn_body(qsel_ref, k_ref, v_ref, o_ref):
    for h in range(_H):
        q = _bf16(qsel_ref[h])
        kh = _bf16(k_ref[:, h * _HD:(h + 1) * _HD])
        s = lax.dot_general(q, kh, (((1,), (1,)), ((), ())),
                            preferred_element_type=jnp.float32) * _SCALE
        m = jnp.max(s, axis=1, keepdims=True)
        e = jnp.exp(s - m)
        p = e / jnp.sum(e, axis=1, keepdims=True)
        vh = _bf16(v_ref[:, h * _HD:(h + 1) * _HD])
        o_ref[h] = lax.dot_general(_bf16(p), vh, (((1,), (0,)), ((), ())),
                                   preferred_element_type=jnp.float32)


def _attention(qsel, k2, v2):
    return pl.pallas_call(
        _attn_body,
        grid=(_B,),
        in_specs=[
            pl.BlockSpec((_H, _UP, _HD), lambda i: (i, 0, 0)),
            pl.BlockSpec((_L, _D), lambda i: (i, 0)),
            pl.BlockSpec((_L, _D), lambda i: (i, 0)),
        ],
        out_specs=pl.BlockSpec((_H, _UP, _HD), lambda i: (i, 0, 0)),
        out_shape=jax.ShapeDtypeStruct((_BH, _UP, _HD), jnp.float32),
    )(qsel, k2, v2)


def _scatter(osel, dsel, dcomp4, zrows):
    mesh = plsc.VectorSubcoreMesh(core_axis_name="c", subcore_axis_name="s")

    @functools.partial(
        pl.kernel,
        mesh=mesh,
        out_type=jax.ShapeDtypeStruct((_B * _L * _H, _HD), jnp.float32),
        scratch_types=[
            pltpu.VMEM((_UP, _HD), jnp.float32),
            pltpu.VMEM((_CH, _HD), jnp.float32),
            pltpu.VMEM((4, _CH), jnp.int32),
            pltpu.VMEM((16, _CH), jnp.int32),
            pltpu.SemaphoreType.DMA,
        ],
        compiler_params=pltpu.CompilerParams(use_tc_tiling_on_sc=False,
                                             needs_layout_passes=False),
    )
    def scat_kernel(osel_hbm, dsel_hbm, dcomp_hbm, z_hbm, out_hbm,
                    ov, zv, dv, cv, sem):
        cid = lax.axis_index("c")
        sid = lax.axis_index("s")
        w = cid * 16 + sid
        pltpu.sync_copy(osel_hbm.at[w], ov)
        pltpu.sync_copy(z_hbm, zv)
        pltpu.sync_copy(dsel_hbm.at[pl.ds(w * 4, 4)], dv)
        pltpu.sync_copy(dcomp_hbm.at[pl.ds(w * 16, 16)], cv)
        for j in range(4):
            pltpu.async_copy(ov.at[pl.ds(j * _CH, _CH)],
                             out_hbm.at[dv.at[j]], sem).wait()
        for j in range(16):
            pltpu.async_copy(zv, out_hbm.at[cv.at[j]], sem).wait()

    return scat_kernel(osel, dsel, dcomp4, zrows)


def _proj_body(y_ref, wo_ref, bo_ref, o_ref):
    o_ref[...] = lax.dot_general(
        _bf16(y_ref[...]), _bf16(wo_ref[...]), (((1,), (0,)), ((), ())),
        preferred_element_type=jnp.float32) + bo_ref[...]


def _proj(y2, Wo, bo):
    return pl.pallas_call(
        _proj_body,
        grid=(_NRT,),
        in_specs=[
            pl.BlockSpec((_RT, _D), lambda i: (i, 0)),
            pl.BlockSpec((_D, _D), lambda i: (0, 0)),
            pl.BlockSpec((1, _D), lambda i: (0, 0)),
        ],
        out_specs=pl.BlockSpec((_RT, _D), lambda i: (i, 0)),
        out_shape=jax.ShapeDtypeStruct((_B * _L, _D), jnp.float32),
    )(y2, Wo, bo.reshape(1, _D))


def kernel(x, Wq, bq, Wk, bk, Wv, bv, Wo, bo):
    x2 = x.reshape(_B * _L, _D)
    q2, norms = _qn(x2, Wq, bq)
    qflat = q2.reshape(_B * _L * _H, _HD)
    qsel, dsel, dcomp = _select_gather(norms, qflat)
    k2, v2 = _kv(x2, Wk, Wv, bk, bv)
    osel = _attention(qsel, k2, v2)
    zrows = jnp.zeros((_CH, _HD), jnp.float32)
    outflat = _scatter(osel, dsel.reshape(_BH * 4, _CH),
                       dcomp.reshape(_BH * 16, _CH), zrows)
    out2 = _proj(outflat.reshape(_B * _L, _D), Wo, bo)
    return out2.reshape(_B, _L, _D)
